# baseline (speedup 1.0000x reference)
import jax
import jax.numpy as jnp
from jax.experimental import pallas as pl


_BLK = 4000


def _encoder_body(x_ref, tk_ref, ct_ref, wr1_ref, b1_ref, wr2_ref, b2_ref,
                  wrt_ref, brt_ref, wrc_ref, brc_ref,
                  out_li_ref, out_tk_ref, out_ct_ref):
    h = jnp.dot(x_ref[...], wr1_ref[...], preferred_element_type=jnp.float32)
    h = jnp.maximum(h + b1_ref[...], 0.0)
    out_li_ref[...] = (
        jnp.dot(h, wr2_ref[...], preferred_element_type=jnp.float32)
        + b2_ref[...])

    @pl.when(pl.program_id(0) == 0)
    def _tiny():
        out_tk_ref[...] = jnp.maximum(tk_ref[...] * wrt_ref[...] + brt_ref[...], 0.0)
        out_ct_ref[...] = jnp.maximum(ct_ref[...] * wrc_ref[...] + brc_ref[...], 0.0)


def kernel(line_item_x, timekeeper_x, case_type_x, W_l1, b_l1, W_r1,
           W_l2, b_l2, W_r2, W_lt, b_lt, W_rt, W_lc, b_lc, W_rc, edge_index):
    n_li, d_in = line_item_x.shape
    n_tk = timekeeper_x.shape[0]
    n_ct = case_type_x.shape[0]
    d_h = W_r1.shape[0]
    d_e = W_r2.shape[0]

    wr1t = W_r1.T
    wr2t = W_r2.T
    b1 = b_l1.reshape(1, d_h)
    b2 = b_l2.reshape(1, d_e)
    wrt = W_rt.reshape(1, d_h)
    brt = b_lt.reshape(1, d_h)
    wrc = W_rc.reshape(1, d_h)
    brc = b_lc.reshape(1, d_h)

    grid = (n_li // _BLK,)

    def fixed(shape):
        nd = len(shape)
        return pl.BlockSpec(shape, lambda i, _n=nd: (0,) * _n)

    out_li, out_tk, out_ct = pl.pallas_call(
        _encoder_body,
        grid=grid,
        in_specs=[
            pl.BlockSpec((_BLK, d_in), lambda i: (i, 0)),
            fixed((n_tk, 1)),
            fixed((n_ct, 1)),
            fixed((d_in, d_h)),
            fixed((1, d_h)),
            fixed((d_h, d_e)),
            fixed((1, d_e)),
            fixed((1, d_h)),
            fixed((1, d_h)),
            fixed((1, d_h)),
            fixed((1, d_h)),
        ],
        out_specs=[
            pl.BlockSpec((_BLK, d_e), lambda i: (i, 0)),
            fixed((n_tk, d_h)),
            fixed((n_ct, d_h)),
        ],
        out_shape=[
            jax.ShapeDtypeStruct((n_li, d_e), jnp.float32),
            jax.ShapeDtypeStruct((n_tk, d_h), jnp.float32),
            jax.ShapeDtypeStruct((n_ct, d_h), jnp.float32),
        ],
    )(line_item_x, timekeeper_x, case_type_x, wr1t, b1, wr2t, b2,
      wrt, brt, wrc, brc)

    return (out_li, out_tk, out_ct)
